# baseline (speedup 1.0000x reference)
import jax
import jax.numpy as jnp
from jax.experimental import pallas as pl
from jax.experimental.pallas import tpu as pltpu

_BM = 200


def _body(f1, f2, f3, adj, w1, w2, w3, womega, urow, wd1, wd2, wd3,
          emb_out, r1, r2, r3, h_scr, emb_scr, embbf_scr):
    p = pl.program_id(0)
    i = pl.program_id(1)
    bm = adj.shape[0]
    f32 = jnp.float32
    b16 = jnp.bfloat16

    @pl.when(p == 0)
    def _proj():
        h1 = jnp.dot(f1[...].astype(b16), w1[...], preferred_element_type=f32)
        h2 = jnp.dot(f2[...].astype(b16), w2[...], preferred_element_type=f32)
        h3 = jnp.dot(f3[...].astype(b16), w3[...], preferred_element_type=f32)
        h_scr[pl.ds(i * bm, bm), :] = jnp.concatenate([h1, h2, h3], axis=1).astype(b16)

    @pl.when(p == 1)
    def _fuse():
        a = adj[...].astype(b16)
        z = jnp.dot(a, h_scr[...], preferred_element_type=f32)
        dz = womega.shape[0]
        zs = [z[:, k * dz:(k + 1) * dz] for k in range(3)]
        w = womega[...]
        u = urow[...]
        ss = []
        for zk in zs:
            v = jnp.tanh(jnp.dot(zk.astype(b16), w, preferred_element_type=f32))
            ss.append(jnp.sum(v * u, axis=1, keepdims=True))
        m = jnp.maximum(jnp.maximum(ss[0], ss[1]), ss[2])
        es = [jnp.exp(s - m) for s in ss]
        den = es[0] + es[1] + es[2]
        emb = (es[0] * zs[0] + es[1] * zs[1] + es[2] * zs[2]) / den
        emb_scr[pl.ds(i * bm, bm), :] = emb
        embbf_scr[pl.ds(i * bm, bm), :] = emb.astype(b16)

    @pl.when(p == 2)
    def _dec():
        a = adj[...].astype(b16)
        ae = jnp.dot(a, embbf_scr[...], preferred_element_type=f32)
        aeb = ae.astype(b16)
        r1[...] = jnp.dot(aeb, wd1[...], preferred_element_type=f32)
        r2[...] = jnp.dot(aeb, wd2[...], preferred_element_type=f32)
        r3[...] = jnp.dot(aeb, wd3[...], preferred_element_type=f32)
        emb_out[...] = emb_scr[pl.ds(i * bm, bm), :]


def kernel(features_omics1, features_omics2, features_omics3, adj,
           W_enc1, W_enc2, W_enc3, W_dec1, W_dec2, W_dec3,
           w_omega, u_omega):
    n, d1 = features_omics1.shape
    d2 = features_omics2.shape[1]
    d3 = features_omics3.shape[1]
    dz = W_enc1.shape[1]
    b16 = jnp.bfloat16
    w1b, w2b, w3b = W_enc1.astype(b16), W_enc2.astype(b16), W_enc3.astype(b16)
    wd1b, wd2b, wd3b = W_dec1.astype(b16), W_dec2.astype(b16), W_dec3.astype(b16)
    wob = w_omega.astype(b16)
    urow = u_omega.reshape(1, dz)
    nblk = n // _BM
    last = nblk - 1

    def mov(p, i):
        return (jnp.where(p == 0, i, last), 0)

    def madj(p, i):
        return (jnp.where(p == 0, 0, i), 0)

    def mout(p, i):
        return (jnp.where(p == 2, i, 0), 0)

    def mconst(p, i):
        return (0, 0)

    emb, r1, r2, r3 = pl.pallas_call(
        _body,
        grid=(3, nblk),
        in_specs=[pl.BlockSpec((_BM, d1), mov),
                  pl.BlockSpec((_BM, d2), mov),
                  pl.BlockSpec((_BM, d3), mov),
                  pl.BlockSpec((_BM, n), madj),
                  pl.BlockSpec((d1, dz), mconst),
                  pl.BlockSpec((d2, dz), mconst),
                  pl.BlockSpec((d3, dz), mconst),
                  pl.BlockSpec((dz, dz), mconst),
                  pl.BlockSpec((1, dz), mconst),
                  pl.BlockSpec((dz, d1), mconst),
                  pl.BlockSpec((dz, d2), mconst),
                  pl.BlockSpec((dz, d3), mconst)],
        out_specs=[pl.BlockSpec((_BM, dz), mout),
                   pl.BlockSpec((_BM, d1), mout),
                   pl.BlockSpec((_BM, d2), mout),
                   pl.BlockSpec((_BM, d3), mout)],
        out_shape=[jax.ShapeDtypeStruct((n, dz), jnp.float32),
                   jax.ShapeDtypeStruct((n, d1), jnp.float32),
                   jax.ShapeDtypeStruct((n, d2), jnp.float32),
                   jax.ShapeDtypeStruct((n, d3), jnp.float32)],
        scratch_shapes=[pltpu.VMEM((n, 3 * dz), b16),
                        pltpu.VMEM((n, dz), jnp.float32),
                        pltpu.VMEM((n, dz), b16)],
    )(features_omics1, features_omics2, features_omics3, adj,
      w1b, w2b, w3b, wob, urow, wd1b, wd2b, wd3b)

    return emb, r1, r2, r3


# speedup vs baseline: 1.1256x; 1.1256x over previous
import jax
import jax.numpy as jnp
from jax.experimental import pallas as pl
from jax.experimental.pallas import tpu as pltpu

_BM = 400


def _body(f1, f2, f3, adj, w1, w2, w3, womega, ucol, wd1, wd2, wd3,
          emb_out, r1, r2, r3, h_scr, emb_scr, embbf_scr):
    p = pl.program_id(0)
    i = pl.program_id(1)
    bm = adj.shape[0]
    f32 = jnp.float32
    b16 = jnp.bfloat16

    @pl.when(p == 0)
    def _proj():
        h1 = jnp.dot(f1[...].astype(b16), w1[...].astype(b16), preferred_element_type=f32)
        h2 = jnp.dot(f2[...].astype(b16), w2[...].astype(b16), preferred_element_type=f32)
        h3 = jnp.dot(f3[...].astype(b16), w3[...].astype(b16), preferred_element_type=f32)
        h_scr[pl.ds(i * bm, bm), :] = jnp.concatenate([h1, h2, h3], axis=1).astype(b16)

    @pl.when(p == 1)
    def _fuse():
        a = adj[...].astype(b16)
        z = jnp.dot(a, h_scr[...], preferred_element_type=f32)
        dz = womega.shape[0]
        zs = [z[:, k * dz:(k + 1) * dz] for k in range(3)]
        w = womega[...].astype(b16)
        u = ucol[...].astype(b16)
        ss = []
        for zk in zs:
            v = jnp.tanh(jnp.dot(zk.astype(b16), w, preferred_element_type=f32))
            ss.append(jnp.dot(v.astype(b16), u, preferred_element_type=f32))
        m = jnp.maximum(jnp.maximum(ss[0], ss[1]), ss[2])
        es = [jnp.exp(s - m) for s in ss]
        den = es[0] + es[1] + es[2]
        emb = (es[0] * zs[0] + es[1] * zs[1] + es[2] * zs[2]) / den
        emb_scr[pl.ds(i * bm, bm), :] = emb
        embbf_scr[pl.ds(i * bm, bm), :] = emb.astype(b16)

    @pl.when(p == 2)
    def _dec():
        a = adj[...].astype(b16)
        ae = jnp.dot(a, embbf_scr[...], preferred_element_type=f32)
        aeb = ae.astype(b16)
        r1[...] = jnp.dot(aeb, wd1[...].astype(b16), preferred_element_type=f32)
        r2[...] = jnp.dot(aeb, wd2[...].astype(b16), preferred_element_type=f32)
        r3[...] = jnp.dot(aeb, wd3[...].astype(b16), preferred_element_type=f32)
        emb_out[...] = emb_scr[pl.ds(i * bm, bm), :]


def kernel(features_omics1, features_omics2, features_omics3, adj,
           W_enc1, W_enc2, W_enc3, W_dec1, W_dec2, W_dec3,
           w_omega, u_omega):
    n, d1 = features_omics1.shape
    d2 = features_omics2.shape[1]
    d3 = features_omics3.shape[1]
    dz = W_enc1.shape[1]
    nblk = n // _BM
    last = nblk - 1

    def mov(p, i):
        return (jnp.where(p == 0, i, last), 0)

    def madj(p, i):
        return (jnp.where(p == 0, 0, i), 0)

    def mout(p, i):
        return (jnp.where(p == 2, i, 0), 0)

    def mconst(p, i):
        return (0, 0)

    emb, r1, r2, r3 = pl.pallas_call(
        _body,
        grid=(3, nblk),
        in_specs=[pl.BlockSpec((_BM, d1), mov),
                  pl.BlockSpec((_BM, d2), mov),
                  pl.BlockSpec((_BM, d3), mov),
                  pl.BlockSpec((_BM, n), madj),
                  pl.BlockSpec((d1, dz), mconst),
                  pl.BlockSpec((d2, dz), mconst),
                  pl.BlockSpec((d3, dz), mconst),
                  pl.BlockSpec((dz, dz), mconst),
                  pl.BlockSpec((dz, 1), mconst),
                  pl.BlockSpec((dz, d1), mconst),
                  pl.BlockSpec((dz, d2), mconst),
                  pl.BlockSpec((dz, d3), mconst)],
        out_specs=[pl.BlockSpec((_BM, dz), mout),
                   pl.BlockSpec((_BM, d1), mout),
                   pl.BlockSpec((_BM, d2), mout),
                   pl.BlockSpec((_BM, d3), mout)],
        out_shape=[jax.ShapeDtypeStruct((n, dz), jnp.float32),
                   jax.ShapeDtypeStruct((n, d1), jnp.float32),
                   jax.ShapeDtypeStruct((n, d2), jnp.float32),
                   jax.ShapeDtypeStruct((n, d3), jnp.float32)],
        scratch_shapes=[pltpu.VMEM((n, 3 * dz), jnp.bfloat16),
                        pltpu.VMEM((n, dz), jnp.float32),
                        pltpu.VMEM((n, dz), jnp.bfloat16)],
    )(features_omics1, features_omics2, features_omics3, adj,
      W_enc1, W_enc2, W_enc3, w_omega, u_omega, W_dec1, W_dec2, W_dec3)

    return emb, r1, r2, r3


# speedup vs baseline: 1.1710x; 1.0404x over previous
import jax
import jax.numpy as jnp
from jax.experimental import pallas as pl
from jax.experimental.pallas import tpu as pltpu

_BM = 400
_BP = 2000


def _body(f1, f2, f3, adj, w1, w2, w3, womega, ucol, wd1, wd2, wd3,
          emb_out, r1, r2, r3, h_scr, emb_scr, embbf_scr, *, np_, nblk):
    s = pl.program_id(0)
    bm = adj.shape[0]
    f32 = jnp.float32
    b16 = jnp.bfloat16

    @pl.when(s < np_)
    def _proj():
        h1 = jnp.dot(f1[...].astype(b16), w1[...].astype(b16), preferred_element_type=f32)
        h2 = jnp.dot(f2[...].astype(b16), w2[...].astype(b16), preferred_element_type=f32)
        h3 = jnp.dot(f3[...].astype(b16), w3[...].astype(b16), preferred_element_type=f32)
        h_scr[pl.ds(s * _BP, _BP), :] = jnp.concatenate([h1, h2, h3], axis=1).astype(b16)

    @pl.when((s >= np_) & (s < np_ + nblk))
    def _fuse():
        i = s - np_
        a = adj[...].astype(b16)
        z = jnp.dot(a, h_scr[...], preferred_element_type=f32)
        dz = womega.shape[0]
        zs = [z[:, k * dz:(k + 1) * dz] for k in range(3)]
        w = womega[...].astype(b16)
        u = ucol[...].astype(b16)
        ss = []
        for zk in zs:
            v = jnp.tanh(jnp.dot(zk.astype(b16), w, preferred_element_type=f32))
            ss.append(jnp.dot(v.astype(b16), u, preferred_element_type=f32))
        m = jnp.maximum(jnp.maximum(ss[0], ss[1]), ss[2])
        es = [jnp.exp(x - m) for x in ss]
        den = es[0] + es[1] + es[2]
        emb = (es[0] * zs[0] + es[1] * zs[1] + es[2] * zs[2]) / den
        emb_scr[pl.ds(i * bm, bm), :] = emb
        embbf_scr[pl.ds(i * bm, bm), :] = emb.astype(b16)

    @pl.when(s >= np_ + nblk)
    def _dec():
        i = s - np_ - nblk
        a = adj[...].astype(b16)
        ae = jnp.dot(a, embbf_scr[...], preferred_element_type=f32)
        aeb = ae.astype(b16)
        r1[...] = jnp.dot(aeb, wd1[...].astype(b16), preferred_element_type=f32)
        r2[...] = jnp.dot(aeb, wd2[...].astype(b16), preferred_element_type=f32)
        r3[...] = jnp.dot(aeb, wd3[...].astype(b16), preferred_element_type=f32)
        emb_out[...] = emb_scr[pl.ds(i * bm, bm), :]


def kernel(features_omics1, features_omics2, features_omics3, adj,
           W_enc1, W_enc2, W_enc3, W_dec1, W_dec2, W_dec3,
           w_omega, u_omega):
    import functools
    n, d1 = features_omics1.shape
    d2 = features_omics2.shape[1]
    d3 = features_omics3.shape[1]
    dz = W_enc1.shape[1]
    nblk = n // _BM
    np_ = n // _BP

    def mov(s):
        return (jnp.where(s < np_, s, np_ - 1), 0)

    def madj(s):
        return (jnp.where(s < np_, 0,
                jnp.where(s < np_ + nblk, s - np_, s - np_ - nblk)), 0)

    def mout(s):
        return (jnp.where(s < np_ + nblk, 0, s - np_ - nblk), 0)

    def mconst(s):
        return (0, 0)

    body = functools.partial(_body, np_=np_, nblk=nblk)

    emb, r1, r2, r3 = pl.pallas_call(
        body,
        grid=(np_ + 2 * nblk,),
        in_specs=[pl.BlockSpec((_BP, d1), mov),
                  pl.BlockSpec((_BP, d2), mov),
                  pl.BlockSpec((_BP, d3), mov),
                  pl.BlockSpec((_BM, n), madj),
                  pl.BlockSpec((d1, dz), mconst),
                  pl.BlockSpec((d2, dz), mconst),
                  pl.BlockSpec((d3, dz), mconst),
                  pl.BlockSpec((dz, dz), mconst),
                  pl.BlockSpec((dz, 1), mconst),
                  pl.BlockSpec((dz, d1), mconst),
                  pl.BlockSpec((dz, d2), mconst),
                  pl.BlockSpec((dz, d3), mconst)],
        out_specs=[pl.BlockSpec((_BM, dz), mout),
                   pl.BlockSpec((_BM, d1), mout),
                   pl.BlockSpec((_BM, d2), mout),
                   pl.BlockSpec((_BM, d3), mout)],
        out_shape=[jax.ShapeDtypeStruct((n, dz), jnp.float32),
                   jax.ShapeDtypeStruct((n, d1), jnp.float32),
                   jax.ShapeDtypeStruct((n, d2), jnp.float32),
                   jax.ShapeDtypeStruct((n, d3), jnp.float32)],
        scratch_shapes=[pltpu.VMEM((n, 3 * dz), jnp.bfloat16),
                        pltpu.VMEM((n, dz), jnp.float32),
                        pltpu.VMEM((n, dz), jnp.bfloat16)],
    )(features_omics1, features_omics2, features_omics3, adj,
      W_enc1, W_enc2, W_enc3, w_omega, u_omega, W_dec1, W_dec2, W_dec3)

    return emb, r1, r2, r3


# speedup vs baseline: 1.1740x; 1.0025x over previous
import jax
import jax.numpy as jnp
from jax.experimental import pallas as pl
from jax.experimental.pallas import tpu as pltpu

_BM = 400
_BP = 2000


def _body(f1, f2, f3, adjt, adjb, w1, w2, w3, womega, ucol, wd1, wd2, wd3,
          emb_out, r1, r2, r3, h_scr, emb_scr, embbf_scr, *, np_, nblk):
    s = pl.program_id(0)
    bm = 2 * adjt.shape[0]
    f32 = jnp.float32
    b16 = jnp.bfloat16

    @pl.when(s < np_)
    def _proj():
        h1 = jnp.dot(f1[...].astype(b16), w1[...].astype(b16), preferred_element_type=f32)
        h2 = jnp.dot(f2[...].astype(b16), w2[...].astype(b16), preferred_element_type=f32)
        h3 = jnp.dot(f3[...].astype(b16), w3[...].astype(b16), preferred_element_type=f32)
        h_scr[pl.ds(s * _BP, _BP), :] = jnp.concatenate([h1, h2, h3], axis=1).astype(b16)

    @pl.when((s >= np_) & (s < np_ + nblk))
    def _fuse():
        i = s - np_
        z = jnp.concatenate(
            [jnp.dot(adjt[...].astype(b16), h_scr[...], preferred_element_type=f32),
             jnp.dot(adjb[...].astype(b16), h_scr[...], preferred_element_type=f32)],
            axis=0)
        dz = womega.shape[0]
        zs = [z[:, k * dz:(k + 1) * dz] for k in range(3)]
        w = womega[...].astype(b16)
        u = ucol[...].astype(b16)
        ss = []
        for zk in zs:
            v = jnp.tanh(jnp.dot(zk.astype(b16), w, preferred_element_type=f32))
            ss.append(jnp.dot(v.astype(b16), u, preferred_element_type=f32))
        m = jnp.maximum(jnp.maximum(ss[0], ss[1]), ss[2])
        es = [jnp.exp(x - m) for x in ss]
        den = es[0] + es[1] + es[2]
        emb = (es[0] * zs[0] + es[1] * zs[1] + es[2] * zs[2]) / den
        emb_scr[pl.ds(i * bm, bm), :] = emb
        embbf_scr[pl.ds(i * bm, bm), :] = emb.astype(b16)

    @pl.when(s >= np_ + nblk)
    def _dec():
        i = s - np_ - nblk
        ae = jnp.concatenate(
            [jnp.dot(adjt[...].astype(b16), embbf_scr[...], preferred_element_type=f32),
             jnp.dot(adjb[...].astype(b16), embbf_scr[...], preferred_element_type=f32)],
            axis=0)
        aeb = ae.astype(b16)
        r1[...] = jnp.dot(aeb, wd1[...].astype(b16), preferred_element_type=f32)
        r2[...] = jnp.dot(aeb, wd2[...].astype(b16), preferred_element_type=f32)
        r3[...] = jnp.dot(aeb, wd3[...].astype(b16), preferred_element_type=f32)
        emb_out[...] = emb_scr[pl.ds(i * bm, bm), :]


def kernel(features_omics1, features_omics2, features_omics3, adj,
           W_enc1, W_enc2, W_enc3, W_dec1, W_dec2, W_dec3,
           w_omega, u_omega):
    import functools
    n, d1 = features_omics1.shape
    d2 = features_omics2.shape[1]
    d3 = features_omics3.shape[1]
    dz = W_enc1.shape[1]
    nblk = n // _BM
    np_ = n // _BP

    def mov(s):
        return (jnp.where(s < np_, s, np_ - 1), 0)

    def madj(s):
        return (jnp.where(s < np_, 0,
                jnp.where(s < np_ + nblk, s - np_, s - np_ - nblk)), 0)

    def mout(s):
        return (jnp.where(s < np_ + nblk, 0, s - np_ - nblk), 0)

    def mconst(s):
        return (0, 0)

    body = functools.partial(_body, np_=np_, nblk=nblk)

    emb, r1, r2, r3 = pl.pallas_call(
        body,
        grid=(np_ + 2 * nblk,),
        in_specs=[pl.BlockSpec((_BP, d1), mov),
                  pl.BlockSpec((_BP, d2), mov),
                  pl.BlockSpec((_BP, d3), mov),
                  pl.BlockSpec((_BM // 2, n), lambda s: (2 * madj(s)[0], 0)),
                  pl.BlockSpec((_BM // 2, n), lambda s: (2 * madj(s)[0] + 1, 0)),
                  pl.BlockSpec((d1, dz), mconst),
                  pl.BlockSpec((d2, dz), mconst),
                  pl.BlockSpec((d3, dz), mconst),
                  pl.BlockSpec((dz, dz), mconst),
                  pl.BlockSpec((dz, 1), mconst),
                  pl.BlockSpec((dz, d1), mconst),
                  pl.BlockSpec((dz, d2), mconst),
                  pl.BlockSpec((dz, d3), mconst)],
        out_specs=[pl.BlockSpec((_BM, dz), mout),
                   pl.BlockSpec((_BM, d1), mout),
                   pl.BlockSpec((_BM, d2), mout),
                   pl.BlockSpec((_BM, d3), mout)],
        out_shape=[jax.ShapeDtypeStruct((n, dz), jnp.float32),
                   jax.ShapeDtypeStruct((n, d1), jnp.float32),
                   jax.ShapeDtypeStruct((n, d2), jnp.float32),
                   jax.ShapeDtypeStruct((n, d3), jnp.float32)],
        scratch_shapes=[pltpu.VMEM((n, 3 * dz), jnp.bfloat16),
                        pltpu.VMEM((n, dz), jnp.float32),
                        pltpu.VMEM((n, dz), jnp.bfloat16)],
    )(features_omics1, features_omics2, features_omics3, adj, adj,
      W_enc1, W_enc2, W_enc3, w_omega, u_omega, W_dec1, W_dec2, W_dec3)

    return emb, r1, r2, r3
